# baseline (speedup 1.0000x reference)
import functools

import jax
import jax.numpy as jnp
from jax import lax
from jax.experimental import pallas as pl
from jax.experimental.pallas import tpu as pltpu
from jax.experimental.pallas import tpu_sc as plsc

_B = 5000
_EPS = 1e-6

_N = 50000
_C = 512
_NW = 32
_CH = 40
_BASE = 1520
_EMAX = 1600


def _mlp_t(hT, W1T, b1, W2T, b2, W3T, b3, W4T, b4):
    a1 = jnp.maximum(jnp.dot(W1T, hT, preferred_element_type=jnp.float32) + b1, 0.0)
    a2 = jnp.maximum(jnp.dot(W2T, a1, preferred_element_type=jnp.float32) + b2, 0.0)
    a3 = jnp.maximum(jnp.dot(W3T, a2, preferred_element_type=jnp.float32) + b3, 0.0)
    return jnp.dot(W4T, a3, preferred_element_type=jnp.float32) + b4


def _dist_body(x_ref, keys_ref,
               W1_ref, b1c_ref, b1r_ref, W2_ref, b2c_ref, b2r_ref,
               W3_ref, b3c_ref, b3r_ref, W4_ref, b4c_ref, b4r_ref,
               d_ref, m_ref, s_ref, qc_ref, cc_ref, ms_ref, ss_ref):
    i = pl.program_id(0)
    params = (W1_ref[...], b1c_ref[...], W2_ref[...], b2c_ref[...],
              W3_ref[...], b3c_ref[...], W4_ref[...], b4c_ref[...])

    @pl.when(i == 0)
    def _init():
        dn = (((1,), (1,)), ((), ()))
        a1 = jnp.maximum(jax.lax.dot_general(x_ref[...], W1_ref[...], dn,
                                             preferred_element_type=jnp.float32)
                         + b1r_ref[...], 0.0)
        a2 = jnp.maximum(jax.lax.dot_general(a1, W2_ref[...], dn,
                                             preferred_element_type=jnp.float32)
                         + b2r_ref[...], 0.0)
        a3 = jnp.maximum(jax.lax.dot_general(a2, W3_ref[...], dn,
                                             preferred_element_type=jnp.float32)
                         + b3r_ref[...], 0.0)
        qx = (jax.lax.dot_general(a3, W4_ref[...], dn,
                                  preferred_element_type=jnp.float32)
              + b4r_ref[...])
        qc = qx - _EPS
        qc_ref[0:1, 0:3] = qc
        cc_ref[0] = jnp.sum(qc * qc)
        ms_ref[0] = jnp.float32(3.0e38)
        ss_ref[0] = jnp.float32(0.0)

    kxT = _mlp_t(keys_ref[...].T, *params)
    ssq = jnp.sum(kxT * kxT, axis=0, keepdims=True)
    cdot = jnp.dot(qc_ref[0:1, 0:3], kxT,
                   preferred_element_type=jnp.float32)
    d2 = jnp.maximum(ssq - 2.0 * cdot + cc_ref[0], 0.0)
    d = jnp.sqrt(d2)
    d_ref[...] = d.reshape(1, 1, _B)

    m_old = ms_ref[0]
    m_new = jnp.minimum(m_old, jnp.min(d))
    scale = jnp.exp(m_new - m_old)
    ss_ref[0] = ss_ref[0] * scale + jnp.sum(jnp.exp(m_new - d))
    ms_ref[0] = m_new

    m_ref[...] = jnp.full((1, 128), ms_ref[0], jnp.float32)
    s_ref[...] = jnp.full((1, 128), ss_ref[0], jnp.float32)


def _sc_body(d_hbm, m_hbm, classes_hbm, out_hbm,
             e_v, erep_v, cls_v, acc_v, m_v, sem0, sem1):
    wid = lax.axis_index("c") * 16 + lax.axis_index("s")
    extra = (wid < 17).astype(jnp.int32)
    start = _BASE * wid + 80 * jnp.minimum(wid, 17)
    n_pairs = 19 + extra

    pltpu.sync_copy(m_hbm.at[pl.ds(0, 16)], m_v)

    @pl.when(wid < 17)
    def _():
        pltpu.sync_copy(d_hbm.at[pl.ds(start, _EMAX)], e_v)

    @pl.when(wid >= 17)
    def _():
        pltpu.sync_copy(d_hbm.at[pl.ds(start, _BASE)], e_v.at[pl.ds(0, _BASE)])

    def _copy(chunk, buf):
        return pltpu.make_async_copy(
            classes_hbm.at[pl.ds(start + chunk * _CH, _CH)],
            cls_v.at[buf], sem0 if buf == 0 else sem1)

    _copy(0, 0).start()
    _copy(1, 1).start()

    mvec = m_v[...]
    bidx = [jnp.full((16,), r, jnp.int32) for r in range(16)]

    def conv_body(j, carry):
        e16 = jnp.exp(mvec - e_v[pl.ds(j * 16, 16)])
        for r in range(16):
            erep_v[pl.ds(j * 256 + r * 16, 16)] = e16.at[bidx[r]].get(
                mode="promise_in_bounds")
        return carry

    lax.fori_loop(0, _EMAX // 16, conv_body, 0)

    def _rows(buf_ref, rowbase, acc):
        def row_body(r, acc):
            eb = erep_v[pl.ds((rowbase + r) * 16, 16)]
            return tuple(acc[j] + eb * buf_ref[r, pl.ds(j * 16, 16)]
                         for j in range(_C // 16))
        return lax.fori_loop(0, _CH, row_body, acc)

    def pair_body(g, acc):
        c0 = 2 * g
        _copy(c0, 0).wait()
        acc = _rows(cls_v.at[0], c0 * _CH, acc)

        @pl.when(g + 1 < n_pairs)
        def _():
            _copy(c0 + 2, 0).start()

        _copy(c0 + 1, 1).wait()
        acc = _rows(cls_v.at[1], (c0 + 1) * _CH, acc)

        @pl.when(g + 1 < n_pairs)
        def _():
            _copy(c0 + 3, 1).start()

        return acc

    acc0 = tuple(jnp.zeros((16,), jnp.float32) for _ in range(_C // 16))
    acc = lax.fori_loop(0, n_pairs, pair_body, acc0)

    for j in range(_C // 16):
        acc_v[pl.ds(j * 16, 16)] = acc[j]
    pltpu.sync_copy(acc_v, out_hbm.at[wid])


def _combine_body(p_ref, s_ref, out_ref):
    s = s_ref[0, 0]
    total = jnp.sum(p_ref[...], axis=0, keepdims=True)
    out_ref[...] = jnp.log(total / s + 1e-4)


def kernel(x, keys, classes, W1, b1, W2, b2, W3, b3, W4, b4):
    n, _ = keys.shape
    c = classes.shape[1]
    grid = n // _B
    W1T, W2T, W3T, W4T = W1.T, W2.T, W3.T, W4.T
    b1c, b2c, b3c, b4c = (b.reshape(-1, 1) for b in (b1, b2, b3, b4))
    b1r, b2r, b3r, b4r = (b.reshape(1, -1) for b in (b1, b2, b3, b4))
    full = lambda s: pl.BlockSpec(s, lambda i: (0, 0))
    d3, m_arr, s_arr = pl.pallas_call(
        _dist_body,
        grid=(grid,),
        in_specs=[
            full((1, x.shape[1])),
            pl.BlockSpec((_B, keys.shape[1]), lambda i: (i, 0)),
            full(W1T.shape), full(b1c.shape), full(b1r.shape),
            full(W2T.shape), full(b2c.shape), full(b2r.shape),
            full(W3T.shape), full(b3c.shape), full(b3r.shape),
            full(W4T.shape), full(b4c.shape), full(b4r.shape),
        ],
        out_specs=[
            pl.BlockSpec((1, 1, _B), lambda i: (i, 0, 0)),
            pl.BlockSpec((1, 128), lambda i: (0, 0)),
            pl.BlockSpec((1, 128), lambda i: (0, 0)),
        ],
        out_shape=[
            jax.ShapeDtypeStruct((grid, 1, _B), jnp.float32),
            jax.ShapeDtypeStruct((1, 128), jnp.float32),
            jax.ShapeDtypeStruct((1, 128), jnp.float32),
        ],
        scratch_shapes=[
            pltpu.VMEM((8, 128), jnp.float32),
            pltpu.SMEM((1,), jnp.float32),
            pltpu.SMEM((1,), jnp.float32),
            pltpu.SMEM((1,), jnp.float32),
        ],
    )(x, keys,
      W1T, b1c, b1r, W2T, b2c, b2r, W3T, b3c, b3r, W4T, b4c, b4r)

    sc_kernel = functools.partial(
        pl.kernel,
        out_type=jax.ShapeDtypeStruct((_NW, c), jnp.float32),
        mesh=plsc.VectorSubcoreMesh(core_axis_name="c", subcore_axis_name="s",
                                    num_cores=2, num_subcores=16),
        scratch_types=[
            pltpu.VMEM((_EMAX,), jnp.float32),
            pltpu.VMEM((_EMAX * 16,), jnp.float32),
            pltpu.VMEM((2, _CH, c), jnp.float32),
            pltpu.VMEM((c,), jnp.float32),
            pltpu.VMEM((16,), jnp.float32),
            pltpu.SemaphoreType.DMA,
            pltpu.SemaphoreType.DMA,
        ],
    )(_sc_body)
    partials = sc_kernel(d3.reshape(n), m_arr.reshape(128), classes)

    out = pl.pallas_call(
        _combine_body,
        in_specs=[pl.BlockSpec((_NW, c), lambda: (0, 0)),
                  pl.BlockSpec((1, 128), lambda: (0, 0))],
        out_specs=pl.BlockSpec((1, c), lambda: (0, 0)),
        out_shape=jax.ShapeDtypeStruct((1, c), jnp.float32),
    )(partials, s_arr)
    return out.reshape((c,))


# speedup vs baseline: 1.1837x; 1.1837x over previous
import functools

import jax
import jax.numpy as jnp
from jax import lax
from jax.experimental import pallas as pl
from jax.experimental.pallas import tpu as pltpu
from jax.experimental.pallas import tpu_sc as plsc

_B = 5000
_EPS = 1e-6

_N = 50000
_C = 512
_NW = 32
_CH = 40
_R_TC = 32080
_B2 = 3208
_WROWS = 560
_NPAIRS = 7


def _mlp_t(hT, W1T, b1, W2T, b2, W3T, b3, W4T, b4):
    a1 = jnp.maximum(jnp.dot(W1T, hT, preferred_element_type=jnp.float32) + b1, 0.0)
    a2 = jnp.maximum(jnp.dot(W2T, a1, preferred_element_type=jnp.float32) + b2, 0.0)
    a3 = jnp.maximum(jnp.dot(W3T, a2, preferred_element_type=jnp.float32) + b3, 0.0)
    return jnp.dot(W4T, a3, preferred_element_type=jnp.float32) + b4


def _dist_body(x_ref, keys_ref,
               W1_ref, b1c_ref, b1r_ref, W2_ref, b2c_ref, b2r_ref,
               W3_ref, b3c_ref, b3r_ref, W4_ref, b4c_ref, b4r_ref,
               d_ref, m_ref, s_ref, qc_ref, cc_ref, ms_ref, ss_ref):
    i = pl.program_id(0)
    params = (W1_ref[...], b1c_ref[...], W2_ref[...], b2c_ref[...],
              W3_ref[...], b3c_ref[...], W4_ref[...], b4c_ref[...])

    @pl.when(i == 0)
    def _init():
        dn = (((1,), (1,)), ((), ()))
        a1 = jnp.maximum(jax.lax.dot_general(x_ref[...], W1_ref[...], dn,
                                             preferred_element_type=jnp.float32)
                         + b1r_ref[...], 0.0)
        a2 = jnp.maximum(jax.lax.dot_general(a1, W2_ref[...], dn,
                                             preferred_element_type=jnp.float32)
                         + b2r_ref[...], 0.0)
        a3 = jnp.maximum(jax.lax.dot_general(a2, W3_ref[...], dn,
                                             preferred_element_type=jnp.float32)
                         + b3r_ref[...], 0.0)
        qx = (jax.lax.dot_general(a3, W4_ref[...], dn,
                                  preferred_element_type=jnp.float32)
              + b4r_ref[...])
        qc = qx - _EPS
        qc_ref[0:1, 0:3] = qc
        cc_ref[0] = jnp.sum(qc * qc)
        ms_ref[0] = jnp.float32(3.0e38)
        ss_ref[0] = jnp.float32(0.0)

    kxT = _mlp_t(keys_ref[...].T, *params)
    ssq = jnp.sum(kxT * kxT, axis=0, keepdims=True)
    cdot = jnp.dot(qc_ref[0:1, 0:3], kxT,
                   preferred_element_type=jnp.float32)
    d2 = jnp.maximum(ssq - 2.0 * cdot + cc_ref[0], 0.0)
    d = jnp.sqrt(d2)
    d_ref[...] = d.reshape(1, 1, _B)

    m_old = ms_ref[0]
    m_new = jnp.minimum(m_old, jnp.min(d))
    scale = jnp.exp(m_new - m_old)
    ss_ref[0] = ss_ref[0] * scale + jnp.sum(jnp.exp(m_new - d))
    ms_ref[0] = m_new

    m_ref[...] = jnp.full((1, 128), ms_ref[0], jnp.float32)
    s_ref[...] = jnp.full((1, 128), ss_ref[0], jnp.float32)


def _sc_body(d_hbm, m_hbm, classes_hbm, out_hbm,
             e_v, erep_v, cls_v, acc_v, m_v, sem0, sem1):
    wid = lax.axis_index("c") * 16 + lax.axis_index("s")
    start = _R_TC + _WROWS * wid

    pltpu.sync_copy(m_hbm.at[pl.ds(0, 16)], m_v)
    pltpu.sync_copy(d_hbm.at[pl.ds(start, _WROWS)], e_v)

    def _copy(chunk, buf):
        return pltpu.make_async_copy(
            classes_hbm.at[pl.ds(start + chunk * _CH, _CH)],
            cls_v.at[buf], sem0 if buf == 0 else sem1)

    _copy(0, 0).start()
    _copy(1, 1).start()

    mvec = m_v[...]
    bidx = [jnp.full((16,), r, jnp.int32) for r in range(16)]

    def conv_body(j, carry):
        e16 = jnp.exp(mvec - e_v[pl.ds(j * 16, 16)])
        for r in range(16):
            erep_v[pl.ds(j * 256 + r * 16, 16)] = e16.at[bidx[r]].get(
                mode="promise_in_bounds")
        return carry

    lax.fori_loop(0, _WROWS // 16, conv_body, 0)

    def _rows(buf_ref, rowbase, acc):
        def row_body(r, acc):
            eb = erep_v[pl.ds((rowbase + r) * 16, 16)]
            return tuple(acc[j] + eb * buf_ref[r, pl.ds(j * 16, 16)]
                         for j in range(_C // 16))
        return lax.fori_loop(0, _CH, row_body, acc)

    def pair_body(g, acc):
        c0 = 2 * g
        _copy(c0, 0).wait()
        acc = _rows(cls_v.at[0], c0 * _CH, acc)

        @pl.when(g + 1 < _NPAIRS)
        def _():
            _copy(c0 + 2, 0).start()

        _copy(c0 + 1, 1).wait()
        acc = _rows(cls_v.at[1], (c0 + 1) * _CH, acc)

        @pl.when(g + 1 < _NPAIRS)
        def _():
            _copy(c0 + 3, 1).start()

        return acc

    acc0 = tuple(jnp.zeros((16,), jnp.float32) for _ in range(_C // 16))
    acc = lax.fori_loop(0, _NPAIRS, pair_body, acc0)

    for j in range(_C // 16):
        acc_v[pl.ds(j * 16, 16)] = acc[j]
    pltpu.sync_copy(acc_v, out_hbm.at[wid])


def _reduce_body(d_ref, m_ref, c_ref, out_ref):
    i = pl.program_id(0)
    e = jnp.exp(m_ref[0, 0] - d_ref[...].reshape(1, _B2))
    p = jnp.dot(e, c_ref[...], preferred_element_type=jnp.float32)

    @pl.when(i == 0)
    def _():
        out_ref[...] = p

    @pl.when(i > 0)
    def _():
        out_ref[...] = out_ref[...] + p


def _combine_body(psc_ref, ptc_ref, s_ref, out_ref):
    s = s_ref[0, 0]
    total = jnp.sum(psc_ref[...], axis=0, keepdims=True) + ptc_ref[...]
    out_ref[...] = jnp.log(total / s + 1e-4)


def kernel(x, keys, classes, W1, b1, W2, b2, W3, b3, W4, b4):
    n, _ = keys.shape
    c = classes.shape[1]
    grid = n // _B
    W1T, W2T, W3T, W4T = W1.T, W2.T, W3.T, W4.T
    b1c, b2c, b3c, b4c = (b.reshape(-1, 1) for b in (b1, b2, b3, b4))
    b1r, b2r, b3r, b4r = (b.reshape(1, -1) for b in (b1, b2, b3, b4))
    full = lambda s: pl.BlockSpec(s, lambda i: (0, 0))
    d3, m_arr, s_arr = pl.pallas_call(
        _dist_body,
        grid=(grid,),
        in_specs=[
            full((1, x.shape[1])),
            pl.BlockSpec((_B, keys.shape[1]), lambda i: (i, 0)),
            full(W1T.shape), full(b1c.shape), full(b1r.shape),
            full(W2T.shape), full(b2c.shape), full(b2r.shape),
            full(W3T.shape), full(b3c.shape), full(b3r.shape),
            full(W4T.shape), full(b4c.shape), full(b4r.shape),
        ],
        out_specs=[
            pl.BlockSpec((1, 1, _B), lambda i: (i, 0, 0)),
            pl.BlockSpec((1, 128), lambda i: (0, 0)),
            pl.BlockSpec((1, 128), lambda i: (0, 0)),
        ],
        out_shape=[
            jax.ShapeDtypeStruct((grid, 1, _B), jnp.float32),
            jax.ShapeDtypeStruct((1, 128), jnp.float32),
            jax.ShapeDtypeStruct((1, 128), jnp.float32),
        ],
        scratch_shapes=[
            pltpu.VMEM((8, 128), jnp.float32),
            pltpu.SMEM((1,), jnp.float32),
            pltpu.SMEM((1,), jnp.float32),
            pltpu.SMEM((1,), jnp.float32),
        ],
    )(x, keys,
      W1T, b1c, b1r, W2T, b2c, b2r, W3T, b3c, b3r, W4T, b4c, b4r)

    dflat = d3.reshape(n)

    sc_kernel = functools.partial(
        pl.kernel,
        out_type=jax.ShapeDtypeStruct((_NW, c), jnp.float32),
        mesh=plsc.VectorSubcoreMesh(core_axis_name="c", subcore_axis_name="s",
                                    num_cores=2, num_subcores=16),
        scratch_types=[
            pltpu.VMEM((_WROWS,), jnp.float32),
            pltpu.VMEM((_WROWS * 16,), jnp.float32),
            pltpu.VMEM((2, _CH, c), jnp.float32),
            pltpu.VMEM((c,), jnp.float32),
            pltpu.VMEM((16,), jnp.float32),
            pltpu.SemaphoreType.DMA,
            pltpu.SemaphoreType.DMA,
        ],
    )(_sc_body)
    partials_sc = sc_kernel(dflat, m_arr.reshape(128), classes)

    d_tc = dflat[:_R_TC].reshape(_R_TC // _B2, 1, _B2)
    partial_tc = pl.pallas_call(
        _reduce_body,
        grid=(_R_TC // _B2,),
        in_specs=[
            pl.BlockSpec((1, 1, _B2), lambda i: (i, 0, 0)),
            pl.BlockSpec((1, 128), lambda i: (0, 0)),
            pl.BlockSpec((_B2, c), lambda i: (i, 0)),
        ],
        out_specs=pl.BlockSpec((1, c), lambda i: (0, 0)),
        out_shape=jax.ShapeDtypeStruct((1, c), jnp.float32),
    )(d_tc, m_arr, classes)

    out = pl.pallas_call(
        _combine_body,
        in_specs=[pl.BlockSpec((_NW, c), lambda: (0, 0)),
                  pl.BlockSpec((1, c), lambda: (0, 0)),
                  pl.BlockSpec((1, 128), lambda: (0, 0))],
        out_specs=pl.BlockSpec((1, c), lambda: (0, 0)),
        out_shape=jax.ShapeDtypeStruct((1, c), jnp.float32),
    )(partials_sc, partial_tc, s_arr)
    return out.reshape((c,))
